# baseline (speedup 1.0000x reference)
import functools

import jax
import jax.numpy as jnp
from jax.experimental import pallas as pl
from jax.experimental.pallas import tpu as pltpu

B = 4096
D_IN = 2048
H = 4096
N = 2048
K_UNIVERSE = 512
K_TRADE = 128

BM = 512
BK = 512
NI = B // BM
NK = H // BK

BS = 512

T2_BITS = 30
P_ITERS = -(-T2_BITS // NK)


def _kth_largest_bits(bits, k):
    def body(j, t):
        cand = t | (jnp.int32(1) << (jnp.int32(30) - j))
        cnt = jnp.sum((bits >= cand).astype(jnp.int32), axis=1, keepdims=True)
        return jnp.where(cnt >= k, cand, t)

    t0 = jnp.zeros((bits.shape[0], 1), jnp.int32)
    return jax.lax.fori_loop(0, 31, body, t0)


def _universe_body(vol_ref, spr_ref, uni_ref):
    ratio = vol_ref[...] / (spr_ref[...] + 1e-8)
    rbits = jax.lax.bitcast_convert_type(ratio, jnp.int32)
    t1 = _kth_largest_bits(rbits, K_UNIVERSE)
    uni_ref[...] = (rbits >= t1).astype(jnp.int8)


def _mlp_sel_body(x_ref, w1_ref, b1_ref, w2_ref, b2_ref, uni_ref,
                  out_ref, logits_ref, ls_ref, cbits_ref, t2_ref):
    i = pl.program_id(0)
    k = pl.program_id(1)

    @pl.when(i < NI)
    def _mlp():
        h = jnp.dot(x_ref[...], w1_ref[...],
                    preferred_element_type=jnp.float32)
        h = jnp.maximum(h + b1_ref[...], 0.0)
        contrib = jnp.dot(h, w2_ref[...], preferred_element_type=jnp.float32)

        @pl.when(k == 0)
        def _init():
            logits_ref[...] = contrib

        @pl.when(k > 0)
        def _accum():
            logits_ref[...] += contrib

    @pl.when(i > 0)
    def _search():
        cb = cbits_ref[...]
        t = jnp.where(k == 0, jnp.int32(0), t2_ref[...])

        def body(l, t):
            j = k * P_ITERS + l
            bit = jnp.int32(T2_BITS - 1) - j
            cand = t | (jnp.int32(1) << jnp.maximum(bit, 0))
            cnt = jnp.sum((cb >= cand).astype(jnp.int32), axis=1,
                          keepdims=True)
            return jnp.where((bit >= 0) & (cnt >= K_TRADE), cand, t)

        t = jax.lax.fori_loop(0, P_ITERS, body, t)

        @pl.when(k == NK - 1)
        def _finish_sel():
            sel = jnp.where(cb >= t, ls_ref[...], 0.0)
            denom = jnp.sum(jnp.abs(sel), axis=1, keepdims=True) + 1e-8
            out_ref[...] = sel / denom

        @pl.when(k < NK - 1)
        def _carry():
            t2_ref[...] = t

    @pl.when((i < NI) & (k == NK - 1))
    def _finish_mlp():
        signal_repr = jax.nn.sigmoid(logits_ref[...] + b2_ref[...])
        ls = signal_repr - 0.5
        abits = jax.lax.bitcast_convert_type(jnp.abs(ls), jnp.int32)
        ls_ref[...] = ls
        uni = uni_ref[...].astype(jnp.int32)
        cbits_ref[...] = jnp.where(uni > 0, abits, jnp.int32(-1))


@functools.partial(jax.jit, static_argnames=("interpret",))
def _run(signal_features, volatility, spread, W1, b1, W2, b2,
         interpret=False):
    universe = pl.pallas_call(
        _universe_body,
        grid=(B // BS,),
        in_specs=[
            pl.BlockSpec((BS, N), lambda i: (i, 0)),
            pl.BlockSpec((BS, N), lambda i: (i, 0)),
        ],
        out_specs=pl.BlockSpec((BS, N), lambda i: (i, 0)),
        out_shape=jax.ShapeDtypeStruct((B, N), jnp.int8),
        compiler_params=pltpu.CompilerParams(
            dimension_semantics=("parallel",),
        ),
        interpret=interpret,
    )(volatility, spread)

    action = pl.pallas_call(
        _mlp_sel_body,
        grid=(NI + 1, NK),
        in_specs=[
            pl.BlockSpec((BM, D_IN), lambda i, k: (jnp.minimum(i, NI - 1), 0)),
            pl.BlockSpec((D_IN, BK), lambda i, k: (0, k)),
            pl.BlockSpec((1, BK), lambda i, k: (0, k)),
            pl.BlockSpec((BK, N), lambda i, k: (k, 0)),
            pl.BlockSpec((1, N), lambda i, k: (0, 0)),
            pl.BlockSpec((BM, N), lambda i, k: (jnp.minimum(i, NI - 1), 0)),
        ],
        out_specs=pl.BlockSpec((BM, N), lambda i, k: (jnp.maximum(i - 1, 0), 0)),
        out_shape=jax.ShapeDtypeStruct((B, N), jnp.float32),
        scratch_shapes=[
            pltpu.VMEM((BM, N), jnp.float32),
            pltpu.VMEM((BM, N), jnp.float32),
            pltpu.VMEM((BM, N), jnp.int32),
            pltpu.VMEM((BM, 1), jnp.int32),
        ],
        compiler_params=pltpu.CompilerParams(
            dimension_semantics=("arbitrary", "arbitrary"),
        ),
        interpret=interpret,
    )(signal_features, W1, b1.reshape(1, H), W2, b2.reshape(1, N), universe)
    return action, jnp.zeros_like(action)


def kernel(signal_features, volatility, spread, W1, b1, W2, b2):
    return _run(signal_features, volatility, spread, W1, b1, W2, b2)


# speedup vs baseline: 1.0975x; 1.0975x over previous
import functools

import jax
import jax.numpy as jnp
from jax.experimental import pallas as pl
from jax.experimental.pallas import tpu as pltpu

B = 4096
D_IN = 2048
H = 4096
N = 2048
K_UNIVERSE = 512
K_TRADE = 128

BM = 1024
BK = 512
NI = B // BM
NK = H // BK

BS = 512
NCHAIN = 2


def _mlp_body(x_ref, w1_ref, b1_ref, w2_ref, b2_ref, out_ref):
    k = pl.program_id(1)

    h = jnp.dot(x_ref[...], w1_ref[...], preferred_element_type=jnp.float32)
    h = jnp.maximum(h + b1_ref[...], 0.0)
    contrib = jnp.dot(h, w2_ref[...], preferred_element_type=jnp.float32)

    @pl.when(k == 0)
    def _init():
        out_ref[...] = contrib

    @pl.when(k > 0)
    def _accum():
        out_ref[...] += contrib

    @pl.when(k == NK - 1)
    def _finish():
        out_ref[...] = jax.nn.sigmoid(out_ref[...] + b2_ref[...])


def _kth_largest_bits_multi(bits_list, k):
    def body(j, ts):
        bitv = jnp.int32(1) << (jnp.int32(30) - j)
        out = []
        for bits, t in zip(bits_list, ts):
            cand = t | bitv
            cnt = jnp.sum((bits >= cand).astype(jnp.int32), axis=1,
                          keepdims=True)
            out.append(jnp.where(cnt >= k, cand, t))
        return tuple(out)

    t0 = tuple(jnp.zeros((b.shape[0], 1), jnp.int32) for b in bits_list)
    return jax.lax.fori_loop(0, 31, body, t0)


def _select_body(repr_ref, vol_ref, spr_ref, out_ref):
    hb = BS // NCHAIN
    ls_l, rbits_l, cbits_l = [], [], []

    for c in range(NCHAIN):
        rows = pl.ds(c * hb, hb)
        ls_l.append(repr_ref[rows, :] - 0.5)
        ratio = vol_ref[rows, :] / (spr_ref[rows, :] + 1e-8)
        rbits_l.append(jax.lax.bitcast_convert_type(ratio, jnp.int32))

    t1_l = _kth_largest_bits_multi(rbits_l, K_UNIVERSE)

    for c in range(NCHAIN):
        abits = jax.lax.bitcast_convert_type(jnp.abs(ls_l[c]), jnp.int32)
        cbits_l.append(jnp.where(rbits_l[c] >= t1_l[c], abits,
                                 jnp.int32(-1)))

    t2_l = _kth_largest_bits_multi(cbits_l, K_TRADE)

    for c in range(NCHAIN):
        sel = jnp.where(cbits_l[c] >= t2_l[c], ls_l[c], 0.0)
        denom = jnp.sum(jnp.abs(sel), axis=1, keepdims=True) + 1e-8
        out_ref[pl.ds(c * hb, hb), :] = sel / denom


@functools.partial(jax.jit, static_argnames=("interpret",))
def _run(signal_features, volatility, spread, W1, b1, W2, b2,
         interpret=False):
    signal_repr = pl.pallas_call(
        _mlp_body,
        grid=(NI, NK),
        in_specs=[
            pl.BlockSpec((BM, D_IN), lambda i, k: (i, 0)),
            pl.BlockSpec((D_IN, BK), lambda i, k: (0, k)),
            pl.BlockSpec((1, BK), lambda i, k: (0, k)),
            pl.BlockSpec((BK, N), lambda i, k: (k, 0)),
            pl.BlockSpec((1, N), lambda i, k: (0, 0)),
        ],
        out_specs=pl.BlockSpec((BM, N), lambda i, k: (i, 0)),
        out_shape=jax.ShapeDtypeStruct((B, N), jnp.float32),
        compiler_params=pltpu.CompilerParams(
            dimension_semantics=("parallel", "arbitrary"),
        ),
        interpret=interpret,
    )(signal_features, W1, b1.reshape(1, H), W2, b2.reshape(1, N))

    action = pl.pallas_call(
        _select_body,
        grid=(B // BS,),
        in_specs=[
            pl.BlockSpec((BS, N), lambda i: (i, 0)),
            pl.BlockSpec((BS, N), lambda i: (i, 0)),
            pl.BlockSpec((BS, N), lambda i: (i, 0)),
        ],
        out_specs=pl.BlockSpec((BS, N), lambda i: (i, 0)),
        out_shape=jax.ShapeDtypeStruct((B, N), jnp.float32),
        compiler_params=pltpu.CompilerParams(
            dimension_semantics=("parallel",),
        ),
        interpret=interpret,
    )(signal_repr, volatility, spread)
    return action, jnp.zeros_like(action)


def kernel(signal_features, volatility, spread, W1, b1, W2, b2):
    return _run(signal_features, volatility, spread, W1, b1, W2, b2)
